# baseline (speedup 1.0000x reference)
import functools

import jax
import jax.numpy as jnp
from jax import lax
from jax.experimental import pallas as pl
from jax.experimental.pallas import tpu as pltpu
from jax.experimental.pallas import tpu_sc as plsc

NUM_CLASSES = 100000
FEAT_DIM = 512
BATCH = 16384

NC = 2
NS = 16
LANES = 16
NW = NC * NS
BPW = BATCH // NW
ROW_CHUNK = 64
NCHUNK = BPW // ROW_CHUNK
NVEC = FEAT_DIM // LANES


def _sc_partials(x, labels_i32, centers):
  mesh = plsc.VectorSubcoreMesh(
      core_axis_name="c", subcore_axis_name="s", num_cores=NC,
      num_subcores=NS)

  out_t = jax.ShapeDtypeStruct((BATCH, LANES), jnp.float32)

  @functools.partial(
      pl.kernel,
      out_type=[out_t, out_t, out_t],
      mesh=mesh,
      compiler_params=pltpu.CompilerParams(use_tc_tiling_on_sc=False),
      scratch_types=[
          pltpu.VMEM((ROW_CHUNK,), jnp.int32),
          pltpu.VMEM((ROW_CHUNK, FEAT_DIM), jnp.float32),
          pltpu.VMEM((ROW_CHUNK, FEAT_DIM), jnp.float32),
          pltpu.VMEM((BPW, LANES), jnp.float32),
          pltpu.VMEM((BPW, LANES), jnp.float32),
          pltpu.VMEM((BPW, LANES), jnp.float32),
          pltpu.SemaphoreType.DMA,
      ],
  )
  def sc_kernel(x_hbm, lab_hbm, cen_hbm, outd, outc, outx,
                idx_v, x_v, c_v, ad_v, ac_v, ax_v, sem):
    wid = lax.axis_index("s") * NC + lax.axis_index("c")
    base = wid * BPW
    for k in range(NCHUNK):
      off = base + k * ROW_CHUNK
      pltpu.sync_copy(lab_hbm.at[pl.ds(off, ROW_CHUNK)], idx_v)
      pltpu.sync_copy(x_hbm.at[pl.ds(off, ROW_CHUNK)], x_v)
      pltpu.async_copy(cen_hbm.at[idx_v], c_v, sem).wait()

      def row_body(r, _):
        accd = jnp.zeros((LANES,), jnp.float32)
        accc = jnp.zeros((LANES,), jnp.float32)
        accx = jnp.zeros((LANES,), jnp.float32)
        for v in range(NVEC):
          xv = x_v[r, pl.ds(v * LANES, LANES)]
          cv = c_v[r, pl.ds(v * LANES, LANES)]
          accd = accd + xv * cv
          accc = accc + cv * cv
          accx = accx + xv * xv
        row = k * ROW_CHUNK + r
        ad_v[row, :] = accd
        ac_v[row, :] = accc
        ax_v[row, :] = accx
        return 0

      lax.fori_loop(0, ROW_CHUNK, row_body, 0, unroll=False)

    pltpu.sync_copy(ad_v, outd.at[pl.ds(base, BPW)])
    pltpu.sync_copy(ac_v, outc.at[pl.ds(base, BPW)])
    pltpu.sync_copy(ax_v, outx.at[pl.ds(base, BPW)])

  return sc_kernel(x, labels_i32, centers)


def _tc_finish(dp, cp, xp):
  block = 2048
  grid = BATCH // block

  def body(d_ref, c_ref, x_ref, o_ref):
    i = pl.program_id(0)
    dot = jnp.sum(d_ref[...], axis=1)
    c2 = jnp.sum(c_ref[...], axis=1)
    x2 = jnp.sum(x_ref[...], axis=1)
    cos = dot / jnp.maximum(jnp.sqrt(x2) * jnp.sqrt(c2), 1e-8)
    dist = jnp.exp(-3.5 * (cos - 1.0))
    s = jnp.sum(dist) * (1.0 / BATCH)

    @pl.when(i == 0)
    def _():
      o_ref[0, 0] = 0.0

    o_ref[0, 0] += s

  out = pl.pallas_call(
      body,
      grid=(grid,),
      in_specs=[pl.BlockSpec((block, LANES), lambda i: (i, 0))] * 3,
      out_specs=pl.BlockSpec(memory_space=pltpu.SMEM),
      out_shape=jax.ShapeDtypeStruct((1, 1), jnp.float32),
  )(dp, cp, xp)
  return out[0, 0]


@jax.jit
def kernel(x, labels, centers):
  labels_i32 = labels.astype(jnp.int32)
  dp, cp, xp = _sc_partials(x, labels_i32, centers)
  return _tc_finish(dp, cp, xp)
